# baseline (speedup 1.0000x reference)
import functools

import jax
import jax.numpy as jnp
from jax import lax
from jax.experimental import pallas as pl
from jax.experimental.pallas import tpu as pltpu, tpu_sc as plsc

NUM_FIELDS = 26
VOCAB_P1 = 100001
EMBED_DIM = 16
BATCH = 16384

_INFO = plsc.get_sparse_core_info()
NC, NS, L = _INFO.num_cores, _INFO.num_subcores, _INFO.num_lanes
NW = NC * NS

N = BATCH * NUM_FIELDS
CHUNK = N // NW
G_IDX = 128
GATHERS = CHUNK // G_IDX
BLK_GATHERS = 13
BLK_ROWS = BLK_GATHERS * G_IDX
NBLK = GATHERS // BLK_GATHERS


def _body(x_hbm, tab_hbm, out_hbm, idx_v, rows_v, gsem):
    wid = lax.axis_index("s") * NC + lax.axis_index("c")
    base = wid * CHUNK

    pltpu.sync_copy(x_hbm.at[pl.ds(base, CHUNK)], idx_v)

    lane = lax.broadcasted_iota(jnp.int32, (L,), 0)

    def compute(g, _):
        r = base + g * L + lane
        f = lax.rem(r, NUM_FIELDS)
        idx_v[pl.ds(g * L, L)] = idx_v[pl.ds(g * L, L)] + f * VOCAB_P1
        return 0

    lax.fori_loop(0, CHUNK // L, compute, 0)

    def block(b, _):
        boff = b * BLK_ROWS
        copies = []
        for j in range(BLK_GATHERS):
            copies.append(pltpu.async_copy(
                tab_hbm.at[idx_v.at[pl.ds(boff + j * G_IDX, G_IDX)]],
                rows_v.at[pl.ds(j * G_IDX, G_IDX)],
                gsem,
            ))
        for c in copies:
            c.wait()
        pltpu.sync_copy(rows_v, out_hbm.at[pl.ds(base + boff, BLK_ROWS)])
        return 0

    lax.fori_loop(0, NBLK, block, 0)


_sc_gather = pl.kernel(
    _body,
    out_type=jax.ShapeDtypeStruct((N, EMBED_DIM), jnp.float32),
    mesh=plsc.VectorSubcoreMesh(core_axis_name="c", subcore_axis_name="s"),
    scratch_types=[
        pltpu.VMEM((CHUNK,), jnp.int32),
        pltpu.VMEM((BLK_ROWS, EMBED_DIM), jnp.float32),
        pltpu.SemaphoreType.DMA,
    ],
    compiler_params=pltpu.CompilerParams(use_tc_tiling_on_sc=False),
)


def kernel(x, tables):
    x_flat = x.reshape(-1).astype(jnp.int32)
    tab_flat = tables.reshape(NUM_FIELDS * VOCAB_P1, EMBED_DIM)
    out = _sc_gather(x_flat, tab_flat)
    return out.reshape(BATCH, NUM_FIELDS, EMBED_DIM)


# speedup vs baseline: 1.9740x; 1.9740x over previous
import jax
import jax.numpy as jnp
from jax import lax
from jax.experimental import pallas as pl
from jax.experimental.pallas import tpu as pltpu, tpu_sc as plsc

NUM_FIELDS = 26
VOCAB_P1 = 100001
EMBED_DIM = 16
BATCH = 16384

_INFO = plsc.get_sparse_core_info()
NC, NS, L = _INFO.num_cores, _INFO.num_subcores, _INFO.num_lanes
NW = NC * NS

ROWS_PW = BATCH // NW
G_IDX = 128
GPF = ROWS_PW // G_IDX


def _body(x_hbm, tab_hbm, out_hbm, xv, idxv, rows_v, gsem):
    wid = lax.axis_index("s") * NC + lax.axis_index("c")
    b0 = wid * ROWS_PW

    pltpu.sync_copy(x_hbm.at[pl.ds(b0, ROWS_PW)], xv)

    lane = lax.broadcasted_iota(jnp.int32, (L,), 0)

    def transpose_field(f, _):
        fvec = jnp.full((L,), 0, jnp.int32) + f
        for g in range(ROWS_PW // L):
            col = plsc.load_gather(xv, [g * L + lane, fvec])
            idxv[f, pl.ds(g * L, L)] = col
        return 0

    lax.fori_loop(0, NUM_FIELDS, transpose_field, 0)

    def field(f, _):
        copies = []
        for k in range(GPF):
            copies.append(pltpu.async_copy(
                tab_hbm.at[f].at[idxv.at[f, pl.ds(k * G_IDX, G_IDX)]],
                rows_v.at[pl.ds(k * G_IDX, G_IDX)],
                gsem,
            ))
        for c in copies:
            c.wait()
        pltpu.sync_copy(rows_v, out_hbm.at[pl.ds(b0, ROWS_PW), f])
        return 0

    lax.fori_loop(0, NUM_FIELDS, field, 0)


_sc_lookup = pl.kernel(
    _body,
    out_type=jax.ShapeDtypeStruct((BATCH, NUM_FIELDS, EMBED_DIM), jnp.float32),
    mesh=plsc.VectorSubcoreMesh(core_axis_name="c", subcore_axis_name="s"),
    scratch_types=[
        pltpu.VMEM((ROWS_PW, NUM_FIELDS), jnp.int32),
        pltpu.VMEM((NUM_FIELDS, ROWS_PW), jnp.int32),
        pltpu.VMEM((ROWS_PW, EMBED_DIM), jnp.float32),
        pltpu.SemaphoreType.DMA,
    ],
    compiler_params=pltpu.CompilerParams(
        use_tc_tiling_on_sc=False, needs_layout_passes=False),
)


def kernel(x, tables):
    return _sc_lookup(x.astype(jnp.int32), tables)


# speedup vs baseline: 3.1551x; 1.5983x over previous
import jax
import jax.numpy as jnp
from jax import lax
from jax.experimental import pallas as pl
from jax.experimental.pallas import tpu as pltpu, tpu_sc as plsc

NUM_FIELDS = 26
VOCAB_P1 = 100001
EMBED_DIM = 16
BATCH = 16384

_INFO = plsc.get_sparse_core_info()
NC, NS, L = _INFO.num_cores, _INFO.num_subcores, _INFO.num_lanes
NW = NC * NS

PLANES = NUM_FIELDS * EMBED_DIM
P_PER_W = PLANES // NW


def _body(xT_hbm, tabT_hbm, outT_hbm, idx2, buf0, buf1, g0, g1, o0, o1):
    wid = lax.axis_index("s") * NC + lax.axis_index("c")
    p0 = wid * P_PER_W
    f0 = p0 // EMBED_DIM

    f1 = jnp.minimum(f0 + 1, NUM_FIELDS - 1)
    pltpu.sync_copy(xT_hbm.at[f0], idx2.at[0])
    pltpu.sync_copy(xT_hbm.at[f1], idx2.at[1])

    bufs = (buf0, buf1)
    gsems = (g0, g1)
    osems = (o0, o1)
    pending = [None, None]

    for t in range(P_PER_W):
        slot = t % 2
        p = p0 + t
        f = p // EMBED_DIM
        d = p % EMBED_DIM
        if pending[slot] is not None:
            pending[slot].wait()
        gc = pltpu.async_copy(
            tabT_hbm.at[f].at[d].at[idx2.at[f - f0]],
            bufs[slot],
            gsems[slot],
        )
        gc.wait()
        pending[slot] = pltpu.async_copy(
            bufs[slot], outT_hbm.at[f].at[d], osems[slot])
    for c in pending:
        if c is not None:
            c.wait()


_sc_lookup = pl.kernel(
    _body,
    out_type=jax.ShapeDtypeStruct((NUM_FIELDS, EMBED_DIM, BATCH), jnp.float32),
    mesh=plsc.VectorSubcoreMesh(core_axis_name="c", subcore_axis_name="s"),
    scratch_types=[
        pltpu.VMEM((2, BATCH), jnp.int32),
        pltpu.VMEM((BATCH,), jnp.float32),
        pltpu.VMEM((BATCH,), jnp.float32),
        pltpu.SemaphoreType.DMA,
        pltpu.SemaphoreType.DMA,
        pltpu.SemaphoreType.DMA,
        pltpu.SemaphoreType.DMA,
    ],
    compiler_params=pltpu.CompilerParams(
        use_tc_tiling_on_sc=False, needs_layout_passes=False),
)


def kernel(x, tables):
    xT = x.T.astype(jnp.int32)
    tabT = jnp.swapaxes(tables, 1, 2)
    outT = _sc_lookup(xT, tabT)
    return jnp.transpose(outT, (2, 0, 1))


# speedup vs baseline: 41.9444x; 13.2944x over previous
import jax
import jax.numpy as jnp
from jax import lax
from jax.experimental import pallas as pl
from jax.experimental.pallas import tpu as pltpu, tpu_sc as plsc

NUM_FIELDS = 26
VOCAB_P1 = 100001
EMBED_DIM = 16
BATCH = 16384

_INFO = plsc.get_sparse_core_info()
NC, NS, L = _INFO.num_cores, _INFO.num_subcores, _INFO.num_lanes
NW = NC * NS

PLANES = NUM_FIELDS * EMBED_DIM
P_PER_W = PLANES // NW
HALF = BATCH // 2
UNROLL = 8
GRPS = HALF // (L * UNROLL)


def _body(xT_hbm, tabT_hbm, outT_hbm, plane_v, idx_v, out0, out1, o0, o1):
    osems = (o0, o1)
    wid = lax.axis_index("s") * NC + lax.axis_index("c")
    p0 = wid * P_PER_W

    outs = (out0, out1)
    pending = [None, None]

    for t in range(P_PER_W):
        p = p0 + t
        f = p // EMBED_DIM
        d = p % EMBED_DIM
        pltpu.sync_copy(tabT_hbm.at[f, d], plane_v)
        for h in range(2):
            pltpu.sync_copy(xT_hbm.at[f, pl.ds(h * HALF, HALF)], idx_v)
            if pending[h] is not None:
                pending[h].wait()

            def grp(g, _):
                base = g * L * UNROLL
                for k in range(UNROLL):
                    off = base + k * L
                    iv = idx_v[pl.ds(off, L)]
                    outs[h][pl.ds(off, L)] = plsc.load_gather(plane_v, [iv])
                return 0

            lax.fori_loop(0, GRPS, grp, 0)
            pending[h] = pltpu.async_copy(
                outs[h], outT_hbm.at[f, d, pl.ds(h * HALF, HALF)], osems[h])
    for c in pending:
        if c is not None:
            c.wait()


_sc_lookup = pl.kernel(
    _body,
    out_type=jax.ShapeDtypeStruct((NUM_FIELDS, EMBED_DIM, BATCH), jnp.float32),
    mesh=plsc.VectorSubcoreMesh(core_axis_name="c", subcore_axis_name="s"),
    scratch_types=[
        pltpu.VMEM((VOCAB_P1,), jnp.float32),
        pltpu.VMEM((HALF,), jnp.int32),
        pltpu.VMEM((HALF,), jnp.float32),
        pltpu.VMEM((HALF,), jnp.float32),
        pltpu.SemaphoreType.DMA,
        pltpu.SemaphoreType.DMA,
    ],
    compiler_params=pltpu.CompilerParams(
        use_tc_tiling_on_sc=True, needs_layout_passes=False),
)


def kernel(x, tables):
    xT = x.T.astype(jnp.int32)
    tabT = jnp.swapaxes(tables, 1, 2)
    outT = _sc_lookup(xT, tabT)
    return jnp.transpose(outT, (2, 0, 1))


# speedup vs baseline: 48.1971x; 1.1491x over previous
import jax
import jax.numpy as jnp
from jax import lax
from jax.experimental import pallas as pl
from jax.experimental.pallas import tpu as pltpu, tpu_sc as plsc

NUM_FIELDS = 26
VOCAB_P1 = 100001
EMBED_DIM = 16
BATCH = 16384

_INFO = plsc.get_sparse_core_info()
NC, NS, L = _INFO.num_cores, _INFO.num_subcores, _INFO.num_lanes
NW = NC * NS

PLANES = NUM_FIELDS * EMBED_DIM
P_PER_W = PLANES // NW
QUART = BATCH // 4
UNROLL = 8
GRPS = QUART // (L * UNROLL)


def _body(xT_hbm, tabT_hbm, outT_hbm, plane_v, idx_v, out0, out1, o0, o1):
    wid = lax.axis_index("s") * NC + lax.axis_index("c")
    p0 = wid * P_PER_W

    outs = (out0, out1)
    osems = (o0, o1)
    pending = [None, None]

    for t in range(P_PER_W):
        p = p0 + t
        f = p // EMBED_DIM
        d = p % EMBED_DIM

        if t == 0:
            pltpu.sync_copy(xT_hbm.at[f], idx_v)
        else:
            @pl.when(d == 0)
            def _restage(f=f):
                pltpu.sync_copy(xT_hbm.at[f], idx_v)

        pltpu.sync_copy(tabT_hbm.at[f, d], plane_v)

        for q in range(4):
            slot = q % 2
            if pending[slot] is not None:
                pending[slot].wait()

            def grp(g, _, q=q, slot=slot):
                base = g * L * UNROLL
                for k in range(UNROLL):
                    off = base + k * L
                    iv = idx_v[pl.ds(q * QUART + off, L)]
                    outs[slot][pl.ds(off, L)] = plsc.load_gather(
                        plane_v, [iv])
                return 0

            lax.fori_loop(0, GRPS, grp, 0)
            pending[slot] = pltpu.async_copy(
                outs[slot], outT_hbm.at[f, d, pl.ds(q * QUART, QUART)],
                osems[slot])
    for c in pending:
        if c is not None:
            c.wait()


_sc_lookup = pl.kernel(
    _body,
    out_type=jax.ShapeDtypeStruct((NUM_FIELDS, EMBED_DIM, BATCH), jnp.float32),
    mesh=plsc.VectorSubcoreMesh(core_axis_name="c", subcore_axis_name="s"),
    scratch_types=[
        pltpu.VMEM((VOCAB_P1,), jnp.float32),
        pltpu.VMEM((BATCH,), jnp.int32),
        pltpu.VMEM((QUART,), jnp.float32),
        pltpu.VMEM((QUART,), jnp.float32),
        pltpu.SemaphoreType.DMA,
        pltpu.SemaphoreType.DMA,
    ],
    compiler_params=pltpu.CompilerParams(
        use_tc_tiling_on_sc=True, needs_layout_passes=False),
)


def kernel(x, tables):
    xT = x.T.astype(jnp.int32)
    tabT = jnp.swapaxes(tables, 1, 2)
    outT = _sc_lookup(xT, tabT)
    return jnp.transpose(outT, (2, 0, 1))
